# baseline (speedup 1.0000x reference)
import functools

import jax
import jax.numpy as jnp
from jax import lax
from jax.experimental import pallas as pl
from jax.experimental.pallas import tpu as pltpu
from jax.experimental.pallas import tpu_sc as plsc

T = 2048
D = 1024
H = 2048
NE = 8
CAP = 768
SLOTS = NE * CAP
NCHUNK = 8
DC = D // NCHUNK
BUFROWS = 6272
ZROWS = BUFROWS // 16
ORROWS = SLOTS // 16
TPT = T // 16
TPW = T // 32
BH = 512


def _route_body(x_ref, g_ref, gw_ref, xn_ref, da1_ref, da2_ref,
                ga1_ref, ga2_ref, cw1_ref, cw2_ref):
    x = x_ref[...]
    ssq = jnp.sum(x * x, axis=1, keepdims=True)
    xn = x * lax.rsqrt(ssq / D + 1e-6) * g_ref[...]
    for j in range(NCHUNK):
        xn_ref[j] = xn[:, j * DC:(j + 1) * DC]
    logits = lax.dot_general(xn, gw_ref[...], (((1,), (1,)), ((), ())),
                             preferred_element_type=jnp.float32)
    m = jnp.max(logits, axis=1, keepdims=True)
    ex = jnp.exp(logits - m)
    probs = ex / jnp.sum(ex, axis=1, keepdims=True)
    iota = lax.broadcasted_iota(jnp.int32, (T, NE), 1)
    p1 = jnp.max(probs, axis=1, keepdims=True)
    e1 = jnp.min(jnp.where(probs == p1, iota, NE), axis=1, keepdims=True)
    probs2 = jnp.where(iota == e1, -1.0, probs)
    p2 = jnp.max(probs2, axis=1, keepdims=True)
    e2 = jnp.min(jnp.where(probs2 == p2, iota, NE), axis=1, keepdims=True)
    denom = p1 + p2 + 1e-10
    w1v = p1 / denom
    w2v = p2 / denom
    oh1 = (iota == e1).astype(jnp.int32)
    oh2 = (iota == e2).astype(jnp.int32)

    def cumsum0(a):
        s = 1
        while s < T:
            a = a + jnp.concatenate(
                [jnp.zeros((s, NE), jnp.int32), a[:T - s]], axis=0)
            s *= 2
        return a

    pos1 = jnp.sum(cumsum0(oh1) * oh1, axis=1, keepdims=True) - 1
    pos2 = jnp.sum(cumsum0(oh2) * oh2, axis=1, keepdims=True) - 1
    ok1 = pos1 < CAP
    ok2 = pos2 < CAP
    da1_ref[...] = jnp.where(ok1, e1 * CAP + pos1, SLOTS)
    da2_ref[...] = jnp.where(ok2, e2 * CAP + pos2, SLOTS)
    ga1_ref[...] = jnp.where(ok1, e1 * CAP + pos1, 0)
    ga2_ref[...] = jnp.where(ok2, e2 * CAP + pos2, 0)
    cw1_ref[...] = jnp.where(ok1, w1v, 0.0)
    cw2_ref[...] = jnp.where(ok2, w2v, 0.0)


_route = pl.pallas_call(
    _route_body,
    out_shape=[
        jax.ShapeDtypeStruct((NCHUNK, T, DC), jnp.float32),
        jax.ShapeDtypeStruct((T, 1), jnp.int32),
        jax.ShapeDtypeStruct((T, 1), jnp.int32),
        jax.ShapeDtypeStruct((T, 1), jnp.int32),
        jax.ShapeDtypeStruct((T, 1), jnp.int32),
        jax.ShapeDtypeStruct((T, 1), jnp.float32),
        jax.ShapeDtypeStruct((T, 1), jnp.float32),
    ],
)


@functools.cache
def _make_dispatch():
    mesh = plsc.VectorSubcoreMesh(core_axis_name="c", subcore_axis_name="s")

    @functools.partial(
        pl.kernel,
        mesh=mesh,
        out_type=jax.ShapeDtypeStruct((NCHUNK * SLOTS, DC), jnp.float32),
        scratch_types=[
            pltpu.VMEM((TPT, DC), jnp.float32),
            pltpu.VMEM((TPT,), jnp.int32),
            pltpu.VMEM((TPT,), jnp.int32),
            pltpu.VMEM_SHARED((BUFROWS, DC), jnp.float32),
        ],
    )
    def _dispatch(xn_hbm, da1_hbm, da2_hbm, zeros_hbm, ei_hbm,
                  rows_v, idx1_v, idx2_v, shared):
        c = lax.axis_index("c")
        s = lax.axis_index("s")
        base_t = s * TPT
        pltpu.sync_copy(da1_hbm.at[pl.ds(base_t, TPT)], idx1_v)
        pltpu.sync_copy(da2_hbm.at[pl.ds(base_t, TPT)], idx2_v)
        for cj in range(NCHUNK // 2):
            j = c * (NCHUNK // 2) + cj
            pltpu.sync_copy(zeros_hbm, shared.at[pl.ds(s * ZROWS, ZROWS)])
            pltpu.sync_copy(xn_hbm.at[pl.ds(j * T + base_t, TPT)], rows_v)
            plsc.subcore_barrier()
            pltpu.sync_copy(rows_v, shared.at[idx1_v], add=True)
            pltpu.sync_copy(rows_v, shared.at[idx2_v], add=True)
            plsc.subcore_barrier()
            pltpu.sync_copy(shared.at[pl.ds(s * ORROWS, ORROWS)],
                            ei_hbm.at[pl.ds(j * SLOTS + s * ORROWS, ORROWS)])
            plsc.subcore_barrier()

    return _dispatch


def _ffn_body(ei_ref, w1_ref, w2_ref, w3_ref, out_ref):
    hb = pl.program_id(1)
    h1 = sum(lax.dot_general(ei_ref[j], w1_ref[0, j], (((1,), (0,)), ((), ())),
                             preferred_element_type=jnp.float32)
             for j in range(NCHUNK))
    h2 = sum(lax.dot_general(ei_ref[j], w2_ref[0, j], (((1,), (0,)), ((), ())),
                             preferred_element_type=jnp.float32)
             for j in range(NCHUNK))
    h = h1 * (1.0 / (1.0 + jnp.exp(-h1))) * h2
    part = lax.dot_general(h, w3_ref[0], (((1,), (0,)), ((), ())),
                           preferred_element_type=jnp.float32)

    @pl.when(hb == 0)
    def _():
        out_ref[...] = part

    @pl.when(hb != 0)
    def _():
        out_ref[...] += part


_ffn = pl.pallas_call(
    _ffn_body,
    grid=(NE, H // BH),
    in_specs=[
        pl.BlockSpec((NCHUNK, CAP, DC), lambda e, hb: (0, e, 0)),
        pl.BlockSpec((1, NCHUNK, DC, BH), lambda e, hb: (e, 0, 0, hb)),
        pl.BlockSpec((1, NCHUNK, DC, BH), lambda e, hb: (e, 0, 0, hb)),
        pl.BlockSpec((1, BH, D), lambda e, hb: (e, hb, 0)),
    ],
    out_specs=pl.BlockSpec((CAP, D), lambda e, hb: (e, 0)),
    out_shape=jax.ShapeDtypeStruct((SLOTS, D), jnp.float32),
)


@functools.cache
def _make_combine_gather():
    mesh = plsc.VectorSubcoreMesh(core_axis_name="c", subcore_axis_name="s")

    @functools.partial(
        pl.kernel,
        mesh=mesh,
        out_type=jax.ShapeDtypeStruct((2 * T, D), jnp.float32),
        scratch_types=[
            pltpu.VMEM((TPW,), jnp.int32),
            pltpu.VMEM((TPW, D), jnp.float32),
            pltpu.SemaphoreType.DMA,
        ],
    )
    def _combine_gather(eo_hbm, ga1_hbm, ga2_hbm, g_hbm, idx_v, rows_v, sem):
        c = lax.axis_index("c")
        s = lax.axis_index("s")
        wid = s * 2 + c
        base = wid * TPW
        pltpu.sync_copy(ga1_hbm.at[pl.ds(base, TPW)], idx_v)
        pltpu.async_copy(eo_hbm.at[idx_v], rows_v, sem).wait()
        pltpu.sync_copy(rows_v, g_hbm.at[pl.ds(base, TPW)])
        pltpu.sync_copy(ga2_hbm.at[pl.ds(base, TPW)], idx_v)
        pltpu.async_copy(eo_hbm.at[idx_v], rows_v, sem).wait()
        pltpu.sync_copy(rows_v, g_hbm.at[pl.ds(T + base, TPW)])

    return _combine_gather


def _wadd_body(g_ref, cw1_ref, cw2_ref, out_ref):
    out_ref[...] = cw1_ref[...] * g_ref[0] + cw2_ref[...] * g_ref[1]


_wadd = pl.pallas_call(
    _wadd_body,
    out_shape=jax.ShapeDtypeStruct((T, D), jnp.float32),
)


def kernel(x, norm_g, gate_w, w1, w2, w3):
    b, s, d = x.shape
    xf = x.reshape(T, D)
    xn4, da1, da2, ga1, ga2, cw1, cw2 = _route(xf, norm_g.reshape(1, D), gate_w)
    zeros = jnp.zeros((ZROWS, DC), jnp.float32)
    ei = _make_dispatch()(xn4.reshape(NCHUNK * T, DC),
                          da1.reshape(T), da2.reshape(T), zeros)
    eo = _ffn(ei.reshape(NCHUNK, SLOTS, DC),
              w1.reshape(NE, NCHUNK, DC, H),
              w2.reshape(NE, NCHUNK, DC, H),
              w3)
    g = _make_combine_gather()(eo, ga1.reshape(T), ga2.reshape(T))
    out = _wadd(g.reshape(2, T, D), cw1, cw2)
    return out.reshape(b, s, d)
